# baseline (speedup 1.0000x reference)
import functools

import jax
import jax.numpy as jnp
from jax import lax
from jax.experimental import pallas as pl
from jax.experimental.pallas import tpu as pltpu
from jax.experimental.pallas import tpu_sc as plsc

N = 10000
E = 320000
D = 128

NC = 2
NS = 16
NW = NC * NS
CH = 128
G = 16
EPW = -(-E // (NW * CH * G)) * CH * G
K = EPW // CH
KG = K // G
E_PAD = EPW * NW
N_PAD = 10112
R = N_PAD // NS
BN = 1000

_MESH = dict(core_axis_name="c", subcore_axis_name="s", num_cores=NC,
             num_subcores=NS)

KG_SPLIT = (3, 7)
K0 = KG_SPLIT[0] * G
K1 = KG_SPLIT[1] * G
KMAX = max(K0, K1)
E_SKEW = NS * (K0 + K1) * CH


@functools.lru_cache(maxsize=None)
def _sc_agg():
    @functools.partial(
        pl.kernel,
        out_type=jax.ShapeDtypeStruct((NC, N_PAD, D), jnp.float32),
        mesh=plsc.VectorSubcoreMesh(**_MESH),
        scratch_types=[
            pltpu.VMEM((G, CH), jnp.int32),
            pltpu.VMEM((G, CH), jnp.int32),
            pltpu.VMEM((2, CH, D), jnp.float32),
            pltpu.VMEM_SHARED((N_PAD, D), jnp.float32),
            pltpu.SemaphoreType.DMA,
            pltpu.SemaphoreType.DMA,
            pltpu.SemaphoreType.DMA,
            pltpu.SemaphoreType.DMA,
        ],
    )
    def k(table, src_hbm, dst_hbm, znd, agg_out,
          src_v, dst_v, rows_v, agg_sh, g0, g1, s0, s1):
        c = lax.axis_index("c")
        s = lax.axis_index("s")
        w = s * NC + c
        base = s * R
        gsem = (g0, g1)
        ssem = (s0, s1)
        ngroups = jnp.where(c == 0, KG_SPLIT[0], KG_SPLIT[1])

        pltpu.sync_copy(znd.at[pl.ds(base, R)], agg_sh.at[pl.ds(base, R)])
        plsc.subcore_barrier()

        def group(g, carry):
            pltpu.sync_copy(src_hbm.at[w, pl.ds(g * G, G)], src_v)
            pltpu.sync_copy(dst_hbm.at[w, pl.ds(g * G, G)], dst_v)
            gd = [None, None]
            sd = [None, None]
            gd[0] = pltpu.async_copy(table.at[src_v.at[0]], rows_v.at[0],
                                     gsem[0])
            for j in range(G):
                p = j & 1
                gd[p].wait()
                if j + 1 < G:
                    if sd[1 - p] is not None:
                        sd[1 - p].wait()
                    gd[1 - p] = pltpu.async_copy(
                        table.at[src_v.at[j + 1]], rows_v.at[1 - p],
                        gsem[1 - p])
                sd[p] = pltpu.async_copy(rows_v.at[p],
                                         agg_sh.at[dst_v.at[j]],
                                         ssem[p], add=True)
            sd[(G - 1) & 1].wait()
            return carry

        lax.fori_loop(0, ngroups, group, 0)
        plsc.subcore_barrier()

        pltpu.sync_copy(agg_sh.at[pl.ds(base, R)],
                        agg_out.at[c, pl.ds(base, R)])

    return k


@functools.lru_cache(maxsize=None)
def _sc_deg():
    @functools.partial(
        pl.kernel,
        out_type=jax.ShapeDtypeStruct((NC, N_PAD, D), jnp.float32),
        mesh=plsc.VectorSubcoreMesh(**_MESH),
        scratch_types=[
            pltpu.VMEM((G, CH), jnp.int32),
            pltpu.VMEM((CH, D), jnp.float32),
            pltpu.VMEM_SHARED((N_PAD, D), jnp.float32),
            pltpu.SemaphoreType.DMA,
        ],
    )
    def k(dst_hbm, znd, ones_hbm, deg_out, dst_v, ones_v, deg_sh, sem):
        c = lax.axis_index("c")
        s = lax.axis_index("s")
        w = s * NC + c
        base = s * R

        pltpu.sync_copy(znd.at[pl.ds(base, R)], deg_sh.at[pl.ds(base, R)])
        pltpu.sync_copy(ones_hbm, ones_v)
        plsc.subcore_barrier()

        def group(g, carry):
            pltpu.sync_copy(dst_hbm.at[w, pl.ds(g * G, G)], dst_v)
            sds = [pltpu.async_copy(ones_v, deg_sh.at[dst_v.at[j]], sem,
                                    add=True)
                   for j in range(G)]
            for d in sds:
                d.wait()
            return carry

        lax.fori_loop(0, KG, group, 0)
        plsc.subcore_barrier()

        pltpu.sync_copy(deg_sh.at[pl.ds(base, R)],
                        deg_out.at[c, pl.ds(base, R)])

    return k


def _tc_body(agg_ref, deg_ref, h_ref, wn_ref, ws_ref, b_ref, out_ref):
    agg = agg_ref[0] + agg_ref[1]
    deg = deg_ref[0] + deg_ref[1]
    mean = agg / jnp.maximum(deg, 1.0)
    out = (jnp.dot(mean, wn_ref[...], preferred_element_type=jnp.float32)
           + jnp.dot(h_ref[...], ws_ref[...],
                     preferred_element_type=jnp.float32)
           + b_ref[...])
    out_ref[...] = jnp.maximum(out, 0.0)


_tc_layer = pl.pallas_call(
    _tc_body,
    grid=(N // BN,),
    in_specs=[
        pl.BlockSpec((NC, BN, D), lambda i: (0, i, 0)),
        pl.BlockSpec((NC, BN, D), lambda i: (0, i, 0)),
        pl.BlockSpec((BN, D), lambda i: (i, 0)),
        pl.BlockSpec((D, D), lambda i: (0, 0)),
        pl.BlockSpec((D, D), lambda i: (0, 0)),
        pl.BlockSpec((1, D), lambda i: (0, 0)),
    ],
    out_specs=pl.BlockSpec((BN, D), lambda i: (i, 0)),
    out_shape=jax.ShapeDtypeStruct((N, D), jnp.float32),
)


def _skewed_layout(v):
    c0 = v[:NS * K0 * CH].reshape(NS, K0, CH)
    c1 = v[NS * K0 * CH:].reshape(NS, K1, CH)
    c0p = jnp.concatenate(
        [c0, jnp.zeros((NS, KMAX - K0, CH), jnp.int32)], axis=1)
    return jnp.stack([c0p, c1], axis=1).reshape(NW, KMAX, CH)


def kernel(x, edge_index, W_neigh1, W_self1, b1, W_neigh2, W_self2, b2):
    src = edge_index[0].astype(jnp.int32)
    dst = edge_index[1].astype(jnp.int32)
    pad = E_PAD - E
    dst_r = jnp.concatenate(
        [dst, jnp.full((pad, ), N, jnp.int32)]).reshape(NW, K, CH)
    pad_s = E_SKEW - E
    src_sk = _skewed_layout(
        jnp.concatenate([src, jnp.zeros((pad_s,), jnp.int32)]))
    dst_sk = _skewed_layout(
        jnp.concatenate([dst, jnp.full((pad_s,), N, jnp.int32)]))
    znd = jnp.zeros((N_PAD, D), jnp.float32)
    ones = jnp.ones((CH, D), jnp.float32)
    b1r = b1.reshape(1, D)
    b2r = b2.reshape(1, D)

    degp = _sc_deg()(dst_r, znd, ones)
    agg1 = _sc_agg()(x, src_sk, dst_sk, znd)
    h1 = _tc_layer(agg1, degp, x, W_neigh1, W_self1, b1r)
    agg2 = _sc_agg()(h1, src_sk, dst_sk, znd)
    h2 = _tc_layer(agg2, degp, h1, W_neigh2, W_self2, b2r)
    return h2


# speedup vs baseline: 1.0420x; 1.0420x over previous
import functools

import jax
import jax.numpy as jnp
from jax import lax
from jax.experimental import pallas as pl
from jax.experimental.pallas import tpu as pltpu
from jax.experimental.pallas import tpu_sc as plsc

N = 10000
E = 320000
D = 128

NC = 2
NS = 16
NW = NC * NS
CH = 128
G = 16
EPW = -(-E // (NW * CH * G)) * CH * G
K = EPW // CH
KG = K // G
E_PAD = EPW * NW
N_PAD = 10112
R = N_PAD // NS
BN = 1000

_MESH = dict(core_axis_name="c", subcore_axis_name="s", num_cores=NC,
             num_subcores=NS)

KG_SPLIT = (5, 5)
K0 = KG_SPLIT[0] * G
K1 = KG_SPLIT[1] * G
KMAX = max(K0, K1)
E_SKEW = NS * (K0 + K1) * CH


@functools.lru_cache(maxsize=None)
def _sc_agg():
    @functools.partial(
        pl.kernel,
        out_type=jax.ShapeDtypeStruct((NC, N_PAD, D), jnp.float32),
        mesh=plsc.VectorSubcoreMesh(**_MESH),
        scratch_types=[
            pltpu.VMEM((G, CH), jnp.int32),
            pltpu.VMEM((G, CH), jnp.int32),
            pltpu.VMEM((2, CH, D), jnp.float32),
            pltpu.VMEM_SHARED((N_PAD, D), jnp.float32),
            pltpu.SemaphoreType.DMA,
            pltpu.SemaphoreType.DMA,
            pltpu.SemaphoreType.DMA,
            pltpu.SemaphoreType.DMA,
        ],
    )
    def k(table, src_hbm, dst_hbm, znd, agg_out,
          src_v, dst_v, rows_v, agg_sh, g0, g1, s0, s1):
        c = lax.axis_index("c")
        s = lax.axis_index("s")
        w = s * NC + c
        base = s * R
        gsem = (g0, g1)
        ssem = (s0, s1)
        ngroups = jnp.where(c == 0, KG_SPLIT[0], KG_SPLIT[1])

        pltpu.sync_copy(znd.at[pl.ds(base, R)], agg_sh.at[pl.ds(base, R)])
        plsc.subcore_barrier()

        def group(g, carry):
            pltpu.sync_copy(src_hbm.at[w, pl.ds(g * G, G)], src_v)
            pltpu.sync_copy(dst_hbm.at[w, pl.ds(g * G, G)], dst_v)
            gd = [None, None]
            sd = [None, None]
            gd[0] = pltpu.async_copy(table.at[src_v.at[0]], rows_v.at[0],
                                     gsem[0])
            for j in range(G):
                p = j & 1
                gd[p].wait()
                if j + 1 < G:
                    if sd[1 - p] is not None:
                        sd[1 - p].wait()
                    gd[1 - p] = pltpu.async_copy(
                        table.at[src_v.at[j + 1]], rows_v.at[1 - p],
                        gsem[1 - p])
                sd[p] = pltpu.async_copy(rows_v.at[p],
                                         agg_sh.at[dst_v.at[j]],
                                         ssem[p], add=True)
            sd[(G - 1) & 1].wait()
            return carry

        lax.fori_loop(0, ngroups, group, 0)
        plsc.subcore_barrier()

        pltpu.sync_copy(agg_sh.at[pl.ds(base, R)],
                        agg_out.at[c, pl.ds(base, R)])

    return k


@functools.lru_cache(maxsize=None)
def _sc_deg():
    @functools.partial(
        pl.kernel,
        out_type=jax.ShapeDtypeStruct((NC, N_PAD, D), jnp.float32),
        mesh=plsc.VectorSubcoreMesh(**_MESH),
        scratch_types=[
            pltpu.VMEM((G, CH), jnp.int32),
            pltpu.VMEM((CH, D), jnp.float32),
            pltpu.VMEM_SHARED((N_PAD, D), jnp.float32),
            pltpu.SemaphoreType.DMA,
        ],
    )
    def k(dst_hbm, znd, ones_hbm, deg_out, dst_v, ones_v, deg_sh, sem):
        c = lax.axis_index("c")
        s = lax.axis_index("s")
        w = s * NC + c
        base = s * R

        pltpu.sync_copy(znd.at[pl.ds(base, R)], deg_sh.at[pl.ds(base, R)])
        pltpu.sync_copy(ones_hbm, ones_v)
        plsc.subcore_barrier()

        def group(g, carry):
            pltpu.sync_copy(dst_hbm.at[w, pl.ds(g * G, G)], dst_v)
            sds = [pltpu.async_copy(ones_v, deg_sh.at[dst_v.at[j]], sem,
                                    add=True)
                   for j in range(G)]
            for d in sds:
                d.wait()
            return carry

        lax.fori_loop(0, KG, group, 0)
        plsc.subcore_barrier()

        pltpu.sync_copy(deg_sh.at[pl.ds(base, R)],
                        deg_out.at[c, pl.ds(base, R)])

    return k


def _tc_body(agg_ref, deg_ref, h_ref, wn_ref, ws_ref, b_ref, out_ref):
    agg = agg_ref[0] + agg_ref[1]
    deg = deg_ref[0] + deg_ref[1]
    mean = agg / jnp.maximum(deg, 1.0)
    out = (jnp.dot(mean, wn_ref[...], preferred_element_type=jnp.float32)
           + jnp.dot(h_ref[...], ws_ref[...],
                     preferred_element_type=jnp.float32)
           + b_ref[...])
    out_ref[...] = jnp.maximum(out, 0.0)


_tc_layer = pl.pallas_call(
    _tc_body,
    grid=(N // BN,),
    in_specs=[
        pl.BlockSpec((NC, BN, D), lambda i: (0, i, 0)),
        pl.BlockSpec((NC, BN, D), lambda i: (0, i, 0)),
        pl.BlockSpec((BN, D), lambda i: (i, 0)),
        pl.BlockSpec((D, D), lambda i: (0, 0)),
        pl.BlockSpec((D, D), lambda i: (0, 0)),
        pl.BlockSpec((1, D), lambda i: (0, 0)),
    ],
    out_specs=pl.BlockSpec((BN, D), lambda i: (i, 0)),
    out_shape=jax.ShapeDtypeStruct((N, D), jnp.float32),
)


def _skewed_layout(v):
    c0 = v[:NS * K0 * CH].reshape(NS, K0, CH)
    c1 = v[NS * K0 * CH:].reshape(NS, K1, CH)
    c0p = jnp.concatenate(
        [c0, jnp.zeros((NS, KMAX - K0, CH), jnp.int32)], axis=1)
    return jnp.stack([c0p, c1], axis=1).reshape(NW, KMAX, CH)


def kernel(x, edge_index, W_neigh1, W_self1, b1, W_neigh2, W_self2, b2):
    src = edge_index[0].astype(jnp.int32)
    dst = edge_index[1].astype(jnp.int32)
    pad = E_PAD - E
    dst_r = jnp.concatenate(
        [dst, jnp.full((pad, ), N, jnp.int32)]).reshape(NW, K, CH)
    pad_s = E_SKEW - E
    src_sk = _skewed_layout(
        jnp.concatenate([src, jnp.zeros((pad_s,), jnp.int32)]))
    dst_sk = _skewed_layout(
        jnp.concatenate([dst, jnp.full((pad_s,), N, jnp.int32)]))
    znd = jnp.zeros((N_PAD, D), jnp.float32)
    ones = jnp.ones((CH, D), jnp.float32)
    b1r = b1.reshape(1, D)
    b2r = b2.reshape(1, D)

    degp = _sc_deg()(dst_r, znd, ones)
    agg1 = _sc_agg()(x, src_sk, dst_sk, znd)
    h1 = _tc_layer(agg1, degp, x, W_neigh1, W_self1, b1r)
    agg2 = _sc_agg()(h1, src_sk, dst_sk, znd)
    h2 = _tc_layer(agg2, degp, h1, W_neigh2, W_self2, b2r)
    return h2


# speedup vs baseline: 1.0711x; 1.0279x over previous
import functools

import jax
import jax.numpy as jnp
from jax import lax
from jax.experimental import pallas as pl
from jax.experimental.pallas import tpu as pltpu
from jax.experimental.pallas import tpu_sc as plsc

N = 10000
E = 320000
D = 128

NC = 2
NS = 16
NW = NC * NS
CH = 128
G = 16
EPW = -(-E // (NW * CH * G)) * CH * G
K = EPW // CH
KG = K // G
E_PAD = EPW * NW
N_PAD = 10112
R = N_PAD // NS
BN = 1000

_MESH = dict(core_axis_name="c", subcore_axis_name="s", num_cores=NC,
             num_subcores=NS)

KG_SPLIT = (5, 5)
K0 = KG_SPLIT[0] * G
K1 = KG_SPLIT[1] * G
KMAX = max(K0, K1)
E_SKEW = NS * (K0 + K1) * CH


@functools.lru_cache(maxsize=None)
def _sc_agg():
    @functools.partial(
        pl.kernel,
        out_type=jax.ShapeDtypeStruct((NC, N_PAD, D), jnp.float32),
        mesh=plsc.VectorSubcoreMesh(**_MESH),
        scratch_types=[
            pltpu.VMEM((G, CH), jnp.int32),
            pltpu.VMEM((G, CH), jnp.int32),
            pltpu.VMEM((2, CH, D), jnp.float32),
            pltpu.VMEM_SHARED((N_PAD, D), jnp.float32),
            pltpu.SemaphoreType.DMA,
            pltpu.SemaphoreType.DMA,
            pltpu.SemaphoreType.DMA,
            pltpu.SemaphoreType.DMA,
        ],
    )
    def k(table, src_hbm, dst_hbm, znd, agg_out,
          src_v, dst_v, rows_v, agg_sh, g0, g1, s0, s1):
        c = lax.axis_index("c")
        s = lax.axis_index("s")
        w = s * NC + c
        base = s * R
        gsem = (g0, g1)
        ssem = (s0, s1)
        ngroups = jnp.where(c == 0, KG_SPLIT[0], KG_SPLIT[1])

        pltpu.sync_copy(znd.at[pl.ds(base, R)], agg_sh.at[pl.ds(base, R)])
        plsc.subcore_barrier()

        def group(g, carry):
            pltpu.sync_copy(src_hbm.at[w, pl.ds(g * G, G)], src_v)
            pltpu.sync_copy(dst_hbm.at[w, pl.ds(g * G, G)], dst_v)
            gd = [None, None]
            sd = [None, None]
            gd[0] = pltpu.async_copy(table.at[src_v.at[0]], rows_v.at[0],
                                     gsem[0])
            for j in range(G):
                p = j & 1
                if j + 1 < G:
                    if sd[1 - p] is not None:
                        sd[1 - p].wait()
                    gd[1 - p] = pltpu.async_copy(
                        table.at[src_v.at[j + 1]], rows_v.at[1 - p],
                        gsem[1 - p])
                gd[p].wait()
                sd[p] = pltpu.async_copy(rows_v.at[p],
                                         agg_sh.at[dst_v.at[j]],
                                         ssem[p], add=True)
            sd[(G - 1) & 1].wait()
            return carry

        lax.fori_loop(0, ngroups, group, 0)
        plsc.subcore_barrier()

        pltpu.sync_copy(agg_sh.at[pl.ds(base, R)],
                        agg_out.at[c, pl.ds(base, R)])

    return k


@functools.lru_cache(maxsize=None)
def _sc_deg():
    @functools.partial(
        pl.kernel,
        out_type=jax.ShapeDtypeStruct((NC, N_PAD, D), jnp.float32),
        mesh=plsc.VectorSubcoreMesh(**_MESH),
        scratch_types=[
            pltpu.VMEM((G, CH), jnp.int32),
            pltpu.VMEM((CH, D), jnp.float32),
            pltpu.VMEM_SHARED((N_PAD, D), jnp.float32),
            pltpu.SemaphoreType.DMA,
        ],
    )
    def k(dst_hbm, znd, ones_hbm, deg_out, dst_v, ones_v, deg_sh, sem):
        c = lax.axis_index("c")
        s = lax.axis_index("s")
        w = s * NC + c
        base = s * R

        pltpu.sync_copy(znd.at[pl.ds(base, R)], deg_sh.at[pl.ds(base, R)])
        pltpu.sync_copy(ones_hbm, ones_v)
        plsc.subcore_barrier()

        def group(g, carry):
            pltpu.sync_copy(dst_hbm.at[w, pl.ds(g * G, G)], dst_v)
            sds = [pltpu.async_copy(ones_v, deg_sh.at[dst_v.at[j]], sem,
                                    add=True)
                   for j in range(G)]
            for d in sds:
                d.wait()
            return carry

        lax.fori_loop(0, KG, group, 0)
        plsc.subcore_barrier()

        pltpu.sync_copy(deg_sh.at[pl.ds(base, R)],
                        deg_out.at[c, pl.ds(base, R)])

    return k


def _tc_body(agg_ref, deg_ref, h_ref, wn_ref, ws_ref, b_ref, out_ref):
    agg = agg_ref[0] + agg_ref[1]
    deg = deg_ref[0] + deg_ref[1]
    mean = agg / jnp.maximum(deg, 1.0)
    out = (jnp.dot(mean, wn_ref[...], preferred_element_type=jnp.float32)
           + jnp.dot(h_ref[...], ws_ref[...],
                     preferred_element_type=jnp.float32)
           + b_ref[...])
    out_ref[...] = jnp.maximum(out, 0.0)


_tc_layer = pl.pallas_call(
    _tc_body,
    grid=(N // BN,),
    in_specs=[
        pl.BlockSpec((NC, BN, D), lambda i: (0, i, 0)),
        pl.BlockSpec((NC, BN, D), lambda i: (0, i, 0)),
        pl.BlockSpec((BN, D), lambda i: (i, 0)),
        pl.BlockSpec((D, D), lambda i: (0, 0)),
        pl.BlockSpec((D, D), lambda i: (0, 0)),
        pl.BlockSpec((1, D), lambda i: (0, 0)),
    ],
    out_specs=pl.BlockSpec((BN, D), lambda i: (i, 0)),
    out_shape=jax.ShapeDtypeStruct((N, D), jnp.float32),
)


def _skewed_layout(v):
    c0 = v[:NS * K0 * CH].reshape(NS, K0, CH)
    c1 = v[NS * K0 * CH:].reshape(NS, K1, CH)
    c0p = jnp.concatenate(
        [c0, jnp.zeros((NS, KMAX - K0, CH), jnp.int32)], axis=1)
    return jnp.stack([c0p, c1], axis=1).reshape(NW, KMAX, CH)


def kernel(x, edge_index, W_neigh1, W_self1, b1, W_neigh2, W_self2, b2):
    src = edge_index[0].astype(jnp.int32)
    dst = edge_index[1].astype(jnp.int32)
    pad = E_PAD - E
    dst_r = jnp.concatenate(
        [dst, jnp.full((pad, ), N, jnp.int32)]).reshape(NW, K, CH)
    pad_s = E_SKEW - E
    src_sk = _skewed_layout(
        jnp.concatenate([src, jnp.zeros((pad_s,), jnp.int32)]))
    dst_sk = _skewed_layout(
        jnp.concatenate([dst, jnp.full((pad_s,), N, jnp.int32)]))
    znd = jnp.zeros((N_PAD, D), jnp.float32)
    ones = jnp.ones((CH, D), jnp.float32)
    b1r = b1.reshape(1, D)
    b2r = b2.reshape(1, D)

    degp = _sc_deg()(dst_r, znd, ones)
    agg1 = _sc_agg()(x, src_sk, dst_sk, znd)
    h1 = _tc_layer(agg1, degp, x, W_neigh1, W_self1, b1r)
    agg2 = _sc_agg()(h1, src_sk, dst_sk, znd)
    h2 = _tc_layer(agg2, degp, h1, W_neigh2, W_self2, b2r)
    return h2
